# baseline (speedup 1.0000x reference)
import functools

import jax
import jax.numpy as jnp
from jax import lax
from jax.experimental import pallas as pl
from jax.experimental.pallas import tpu as pltpu
from jax.experimental.pallas import tpu_sc as plsc

N_USERS = 25000
N_ITEMS = 25000
N = N_USERS + N_ITEMS
DH = 32
N_LAYERS = 3
E = 800000
SUB = 128
BLK = 2
NBLK = 204
SUBS_PER_TILE = BLK * NBLK
NTILES = 16
E_PAD = NTILES * SUBS_PER_TILE * SUB
NROWS = E_PAD // SUB
GB = 2
IB = 3
RPT = N // NTILES
ZCH = 25
NCH = RPT // ZCH
GBYTES = SUB * DH * 4


def _sc_propagate(x0h, col2d, row2d, val2d):
    mesh = plsc.VectorSubcoreMesh(core_axis_name="c", subcore_axis_name="s")

    @functools.partial(
        pl.kernel,
        out_type=[
            jax.ShapeDtypeStruct((3, 2, N, DH), jnp.float32),
            jax.ShapeDtypeStruct((2, N, DH), jnp.float32),
        ],
        mesh=mesh,
        compiler_params=pltpu.CompilerParams(use_tc_tiling_on_sc=False),
        scratch_types=[
            pltpu.VMEM_SHARED((N, DH), jnp.float32),
            pltpu.VMEM((GB, BLK, SUB, DH), jnp.float32),
            pltpu.VMEM((IB * BLK, SUB), jnp.int32),
            pltpu.VMEM((IB * BLK, SUB), jnp.int32),
            pltpu.VMEM((IB * BLK, SUB), jnp.float32),
            pltpu.VMEM((ZCH, DH), jnp.float32),
            pltpu.VMEM((ZCH, DH), jnp.float32),
            pltpu.VMEM((ZCH, DH), jnp.float32),
            pltpu.VMEM((ZCH, DH), jnp.float32),
            pltpu.SemaphoreType.DMA,
            pltpu.SemaphoreType.DMA,
            [pltpu.SemaphoreType.DMA] * GB,
        ],
    )
    def k(x0h_hbm, col_hbm, row_hbm, val_hbm, ring_hbm, fin_hbm,
          accum, gbuf, cbuf, rbuf, vbuf, zbuf, tbuf, t0, t1, isem, gsem, ssem):
        c = lax.axis_index("c")
        s = lax.axis_index("s")
        ebase = s * SUBS_PER_TILE
        abase = s * RPT

        zero16 = jnp.zeros((16,), jnp.float32)

        def fire_idx(t, bank):
            rows = pl.ds(ebase + t * BLK, BLK)
            dst = pl.ds(bank * BLK, BLK)
            pltpu.async_copy(col_hbm.at[rows], cbuf.at[dst], isem)
            pltpu.async_copy(row_hbm.at[rows], rbuf.at[dst], isem)
            pltpu.async_copy(val_hbm.at[rows], vbuf.at[dst], isem)

        def wait_idx(bank):
            dst = pl.ds(bank * BLK, BLK)
            pltpu.make_async_copy(col_hbm.at[pl.ds(0, BLK)], cbuf.at[dst], isem).wait()
            pltpu.make_async_copy(row_hbm.at[pl.ds(0, BLK)], rbuf.at[dst], isem).wait()
            pltpu.make_async_copy(val_hbm.at[pl.ds(0, BLK)], vbuf.at[dst], isem).wait()

        def fire_gather(src_hbm, t, gbank, ibank):
            for r in range(BLK):
                pltpu.async_copy(
                    src_hbm.at[cbuf.at[ibank * BLK + r]], gbuf.at[gbank, r], gsem)

        def wait_gather(gbank):
            for r in range(BLK):
                pltpu.make_async_copy(
                    x0h_hbm.at[0, pl.ds(0, SUB)], gbuf.at[gbank, r], gsem).wait()

        def fire_scatter(gbank, ibank):
            for r in range(BLK):
                pltpu.async_copy(
                    gbuf.at[gbank, r], accum.at[rbuf.at[ibank * BLK + r]],
                    ssem[gbank], add=True)

        def wait_scatter(gbank):
            for r in range(BLK):
                pltpu.make_async_copy(
                    gbuf.at[gbank, r], accum.at[pl.ds(0, SUB)],
                    ssem[gbank]).wait()

        def scale(gbank, ibank):
            for r in range(BLK):
                @pl.loop(0, SUB // 16)
                def _(g, r=r):
                    v16 = vbuf[ibank * BLK + r, pl.ds(g * 16, 16)]
                    for j in range(16):
                        kk = g * 16 + j
                        v = v16[j]
                        gbuf[gbank, r, kk, pl.ds(0, 16)] = (
                            gbuf[gbank, r, kk, pl.ds(0, 16)] * v)
                        gbuf[gbank, r, kk, pl.ds(16, 16)] = (
                            gbuf[gbank, r, kk, pl.ds(16, 16)] * v)

        @pl.loop(0, ZCH)
        def _(i):
            zbuf[i, pl.ds(0, 16)] = zero16
            zbuf[i, pl.ds(16, 16)] = zero16

        @pl.loop(0, NCH)
        def _(i):
            rows = pl.ds(abase + i * ZCH, ZCH)
            pltpu.sync_copy(x0h_hbm.at[c, rows], tbuf)
            pltpu.sync_copy(tbuf, ring_hbm.at[0, c, rows])
            pltpu.sync_copy(zbuf, accum.at[rows])

        plsc.subcore_barrier()

        for l in range(N_LAYERS):
            src = ring_hbm.at[l, c]

            fire_idx(0, 0)
            fire_idx(1, 1)
            wait_idx(0)
            fire_gather(src, 0, 0, 0)

            @pl.loop(0, NBLK, step=6)
            def _(i):
                for u in range(6):
                    t = i + u
                    gb, ib = u % GB, u % IB
                    gb1, ib1 = (u + 1) % GB, (u + 1) % IB
                    ib2 = (u + 2) % IB
                    wait_gather(gb)
                    @pl.when(t + 1 < NBLK)
                    def _():
                        wait_idx(ib1)
                    @pl.when(t >= 1)
                    def _():
                        wait_scatter(gb1)
                    @pl.when(t + 1 < NBLK)
                    def _():
                        fire_gather(src, t + 1, gb1, ib1)
                    @pl.when(t + 2 < NBLK)
                    def _():
                        fire_idx(t + 2, ib2)
                    scale(gb, ib)
                    fire_scatter(gb, ib)

            wait_scatter((NBLK - 1) % GB)
            plsc.subcore_barrier()

            if l < 2:
                @pl.loop(0, NCH)
                def _(i):
                    rows = pl.ds(abase + i * ZCH, ZCH)
                    pltpu.sync_copy(accum.at[rows], tbuf)
                    pltpu.sync_copy(tbuf, ring_hbm.at[l + 1, c, rows])
                    pltpu.sync_copy(zbuf, accum.at[rows])

            if l == 2:
                @pl.loop(0, NCH)
                def _(i):
                    rows = pl.ds(abase + i * ZCH, ZCH)
                    pltpu.sync_copy(accum.at[rows], tbuf)
                    pltpu.sync_copy(ring_hbm.at[0, c, rows], t0)
                    pltpu.sync_copy(ring_hbm.at[1, c, rows], t1)

                    @pl.loop(0, ZCH)
                    def _(j):
                        for h in (0, 16):
                            hs = pl.ds(h, 16)
                            tbuf[j, hs] = tbuf[j, hs] + t0[j, hs] + t1[j, hs]
                    pltpu.sync_copy(ring_hbm.at[2, c, rows], t0)

                    @pl.loop(0, ZCH)
                    def _(j):
                        for h in (0, 16):
                            hs = pl.ds(h, 16)
                            tbuf[j, hs] = (tbuf[j, hs] + t0[j, hs]) * 0.25
                    pltpu.sync_copy(tbuf, fin_hbm.at[c, rows])

            plsc.subcore_barrier()

    return k(x0h, col2d, row2d, val2d)


def kernel(edge_index, edge_values, user_emb, item_emb):
    all_emb = jnp.concatenate([user_emb, item_emb], axis=0)
    x0h = jnp.stack([all_emb[:, :DH], all_emb[:, DH:]], axis=0)
    pad = E_PAD - E
    col = jnp.concatenate([edge_index[1], jnp.zeros((pad,), jnp.int32)])
    row = jnp.concatenate([edge_index[0], jnp.zeros((pad,), jnp.int32)])
    val = jnp.concatenate([edge_values, jnp.zeros((pad,), jnp.float32)])
    ring, finalh = _sc_propagate(
        x0h, col.reshape(NROWS, SUB), row.reshape(NROWS, SUB),
        val.reshape(NROWS, SUB))
    del ring
    final = jnp.concatenate([finalh[0], finalh[1]], axis=1)
    return final[:N_USERS], final[N_USERS:]


# speedup vs baseline: 1.0015x; 1.0015x over previous
import functools

import jax
import jax.numpy as jnp
from jax import lax
from jax.experimental import pallas as pl
from jax.experimental.pallas import tpu as pltpu
from jax.experimental.pallas import tpu_sc as plsc

N_USERS = 25000
N_ITEMS = 25000
N = N_USERS + N_ITEMS
DH = 32
N_LAYERS = 3
E = 800000
SUB = 128
BLK = 2
NBLK = 204
SUBS_PER_TILE = BLK * NBLK
NTILES = 16
E_PAD = NTILES * SUBS_PER_TILE * SUB
NROWS = E_PAD // SUB
GB = 2
IB = 3
RPT = N // NTILES
ZCH = 25
NCH = RPT // ZCH
GBYTES = SUB * DH * 4


def _sc_propagate(x0h, col2d, row2d, val2d):
    mesh = plsc.VectorSubcoreMesh(core_axis_name="c", subcore_axis_name="s")

    @functools.partial(
        pl.kernel,
        out_type=[
            jax.ShapeDtypeStruct((3, 2, N, DH), jnp.float32),
            jax.ShapeDtypeStruct((2, N, DH), jnp.float32),
        ],
        mesh=mesh,
        compiler_params=pltpu.CompilerParams(use_tc_tiling_on_sc=False),
        scratch_types=[
            pltpu.VMEM_SHARED((N, DH), jnp.float32),
            pltpu.VMEM((GB, BLK, SUB, DH), jnp.float32),
            pltpu.VMEM((IB * BLK, SUB), jnp.int32),
            pltpu.VMEM((IB * BLK, SUB), jnp.int32),
            pltpu.VMEM((IB * BLK, SUB), jnp.float32),
            pltpu.VMEM((ZCH, DH), jnp.float32),
            pltpu.VMEM((ZCH, DH), jnp.float32),
            pltpu.VMEM((ZCH, DH), jnp.float32),
            pltpu.VMEM((ZCH, DH), jnp.float32),
            pltpu.SemaphoreType.DMA,
            pltpu.SemaphoreType.DMA,
            [pltpu.SemaphoreType.DMA] * GB,
        ],
    )
    def k(x0h_hbm, col_hbm, row_hbm, val_hbm, ring_hbm, fin_hbm,
          accum, gbuf, cbuf, rbuf, vbuf, zbuf, tbuf, t0, t1, isem, gsem, ssem):
        c = lax.axis_index("c")
        s = lax.axis_index("s")
        ebase = s * SUBS_PER_TILE
        abase = s * RPT

        zero16 = jnp.zeros((16,), jnp.float32)

        def fire_idx(t, bank):
            rows = pl.ds(ebase + t * BLK, BLK)
            dst = pl.ds(bank * BLK, BLK)
            pltpu.async_copy(col_hbm.at[rows], cbuf.at[dst], isem)
            pltpu.async_copy(row_hbm.at[rows], rbuf.at[dst], isem)
            pltpu.async_copy(val_hbm.at[rows], vbuf.at[dst], isem)

        def wait_idx(bank):
            dst = pl.ds(bank * BLK, BLK)
            pltpu.make_async_copy(col_hbm.at[pl.ds(0, BLK)], cbuf.at[dst], isem).wait()
            pltpu.make_async_copy(row_hbm.at[pl.ds(0, BLK)], rbuf.at[dst], isem).wait()
            pltpu.make_async_copy(val_hbm.at[pl.ds(0, BLK)], vbuf.at[dst], isem).wait()

        def fire_gather(src_hbm, t, gbank, ibank):
            for r in range(BLK):
                pltpu.async_copy(
                    src_hbm.at[cbuf.at[ibank * BLK + r]], gbuf.at[gbank, r], gsem)

        def wait_gather(gbank):
            for r in range(BLK):
                pltpu.make_async_copy(
                    x0h_hbm.at[0, pl.ds(0, SUB)], gbuf.at[gbank, r], gsem).wait()

        def fire_scatter(gbank, ibank):
            for r in range(BLK):
                pltpu.async_copy(
                    gbuf.at[gbank, r], accum.at[rbuf.at[ibank * BLK + r]],
                    ssem[gbank], add=True)

        def wait_scatter(gbank):
            for r in range(BLK):
                pltpu.make_async_copy(
                    gbuf.at[gbank, r], accum.at[pl.ds(0, SUB)],
                    ssem[gbank]).wait()

        def scale(gbank, ibank):
            for r in range(BLK):
                @pl.loop(0, SUB // 16)
                def _(g, r=r):
                    v16 = vbuf[ibank * BLK + r, pl.ds(g * 16, 16)]
                    for j in range(16):
                        kk = g * 16 + j
                        v = v16[j]
                        gbuf[gbank, r, kk, pl.ds(0, 16)] = (
                            gbuf[gbank, r, kk, pl.ds(0, 16)] * v)
                        gbuf[gbank, r, kk, pl.ds(16, 16)] = (
                            gbuf[gbank, r, kk, pl.ds(16, 16)] * v)

        @pl.loop(0, ZCH)
        def _(i):
            zbuf[i, pl.ds(0, 16)] = zero16
            zbuf[i, pl.ds(16, 16)] = zero16

        @pl.loop(0, NCH)
        def _(i):
            rows = pl.ds(abase + i * ZCH, ZCH)
            pltpu.sync_copy(x0h_hbm.at[c, rows], tbuf)
            pltpu.sync_copy(tbuf, ring_hbm.at[0, c, rows])
            pltpu.sync_copy(zbuf, accum.at[rows])

        plsc.subcore_barrier()

        for l in range(N_LAYERS):
          with jax.named_scope(f"LAYER{l}"):
            src = ring_hbm.at[l, c]

            with jax.named_scope(f"edges{l}"):
              fire_idx(0, 0)
              fire_idx(1, 1)
              wait_idx(0)
              fire_gather(src, 0, 0, 0)

              @pl.loop(0, NBLK, step=6)
              def _(i):
                for u in range(6):
                    t = i + u
                    gb, ib = u % GB, u % IB
                    gb1, ib1 = (u + 1) % GB, (u + 1) % IB
                    ib2 = (u + 2) % IB
                    wait_gather(gb)
                    @pl.when(t + 1 < NBLK)
                    def _():
                        wait_idx(ib1)
                    @pl.when(t >= 1)
                    def _():
                        wait_scatter(gb1)
                    @pl.when(t + 1 < NBLK)
                    def _():
                        fire_gather(src, t + 1, gb1, ib1)
                    @pl.when(t + 2 < NBLK)
                    def _():
                        fire_idx(t + 2, ib2)
                    scale(gb, ib)
                    fire_scatter(gb, ib)

            wait_scatter((NBLK - 1) % GB)
            plsc.subcore_barrier()

            if l < 2:
              with jax.named_scope(f"drain{l}"):
                @pl.loop(0, NCH)
                def _(i):
                    rows = pl.ds(abase + i * ZCH, ZCH)
                    pltpu.sync_copy(accum.at[rows], tbuf)
                    pltpu.sync_copy(tbuf, ring_hbm.at[l + 1, c, rows])
                    pltpu.sync_copy(zbuf, accum.at[rows])

            if l == 2:
              with jax.named_scope("mean"):
                @pl.loop(0, NCH)
                def _(i):
                    rows = pl.ds(abase + i * ZCH, ZCH)
                    pltpu.sync_copy(accum.at[rows], tbuf)
                    pltpu.sync_copy(ring_hbm.at[0, c, rows], t0)
                    pltpu.sync_copy(ring_hbm.at[1, c, rows], t1)

                    @pl.loop(0, ZCH)
                    def _(j):
                        for h in (0, 16):
                            hs = pl.ds(h, 16)
                            tbuf[j, hs] = tbuf[j, hs] + t0[j, hs] + t1[j, hs]
                    pltpu.sync_copy(ring_hbm.at[2, c, rows], t0)

                    @pl.loop(0, ZCH)
                    def _(j):
                        for h in (0, 16):
                            hs = pl.ds(h, 16)
                            tbuf[j, hs] = (tbuf[j, hs] + t0[j, hs]) * 0.25
                    pltpu.sync_copy(tbuf, fin_hbm.at[c, rows])

            plsc.subcore_barrier()

    return k(x0h, col2d, row2d, val2d)


def kernel(edge_index, edge_values, user_emb, item_emb):
    all_emb = jnp.concatenate([user_emb, item_emb], axis=0)
    x0h = jnp.stack([all_emb[:, :DH], all_emb[:, DH:]], axis=0)
    pad = E_PAD - E
    col = jnp.concatenate([edge_index[1], jnp.zeros((pad,), jnp.int32)])
    row = jnp.concatenate([edge_index[0], jnp.zeros((pad,), jnp.int32)])
    val = jnp.concatenate([edge_values, jnp.zeros((pad,), jnp.float32)])
    ring, finalh = _sc_propagate(
        x0h, col.reshape(NROWS, SUB), row.reshape(NROWS, SUB),
        val.reshape(NROWS, SUB))
    del ring
    final = jnp.concatenate([finalh[0], finalh[1]], axis=1)
    return final[:N_USERS], final[N_USERS:]
